# baseline (device time: 24121 ns/iter reference)
import jax
import jax.numpy as jnp
from jax import lax
from jax.experimental import pallas as pl
from jax.experimental.pallas import tpu as pltpu

N_DEV = 16
EPS = 1e-5
N_CHUNKS = 4


def kernel(x, t_emb, W_scale, W_shift):
    b, s, c = x.shape
    c_global = c * N_DEV
    ch = s // N_CHUNKS

    def body(x_hbm, t_ref, ws_ref, wsh_ref, out_hbm,
             x_vmem, obuf, stats_ref, gather_ref,
             in_sems, out_sems, send_sems, recv_sems):
        my = lax.axis_index("i")

        in_copies = []
        for i in range(N_CHUNKS):
            cp = pltpu.make_async_copy(
                x_hbm.at[:, pl.ds(i * ch, ch), :],
                x_vmem.at[:, pl.ds(i * ch, ch), :],
                in_sems.at[i],
            )
            cp.start()
            in_copies.append(cp)

        barrier_sem = pltpu.get_barrier_semaphore()
        for k in range(1, N_DEV):
            pl.semaphore_signal(
                barrier_sem, inc=1,
                device_id=((my + k) % N_DEV,),
                device_id_type=pl.DeviceIdType.MESH,
            )
        pl.semaphore_wait(barrier_sem, N_DEV - 1)

        for i in range(N_CHUNKS):
            in_copies[i].wait()
            xc = x_vmem[:, i * ch:(i + 1) * ch, :]
            stats_ref[0, :, i * ch:(i + 1) * ch] = (
                jnp.sum(xc, axis=-1).astype(jnp.bfloat16))
            stats_ref[1, :, i * ch:(i + 1) * ch] = (
                jnp.sum(xc * xc, axis=-1).astype(jnp.bfloat16))

        rdmas = []
        for k in range(1, N_DEV):
            rdma = pltpu.make_async_remote_copy(
                src_ref=stats_ref,
                dst_ref=gather_ref.at[k - 1],
                send_sem=send_sems.at[k - 1],
                recv_sem=recv_sems.at[k - 1],
                device_id=((my + k) % N_DEV,),
                device_id_type=pl.DeviceIdType.MESH,
            )
            rdma.start()
            rdmas.append(rdma)

        scale = jnp.dot(t_ref[...], ws_ref[...],
                        preferred_element_type=jnp.float32)
        shift = jnp.dot(t_ref[...], wsh_ref[...],
                        preferred_element_type=jnp.float32)
        mul = 1.0 + scale[:, None, :]
        add = shift[:, None, :]

        for rdma in rdmas:
            rdma.wait()

        tot = stats_ref[...].astype(jnp.float32) + jnp.sum(
            gather_ref[...].astype(jnp.float32), axis=0
        )
        mean = tot[0] / c_global
        inv = lax.rsqrt(tot[1] / c_global - mean * mean + EPS)

        out_copies = [None, None]
        for i in range(N_CHUNKS):
            slot = i % 2
            if out_copies[slot] is not None:
                out_copies[slot].wait()
            sl = slice(i * ch, (i + 1) * ch)
            xc = x_vmem[:, sl, :]
            h = (xc - mean[:, sl, None]) * inv[:, sl, None]
            obuf[slot] = (h * mul + add).astype(jnp.bfloat16)
            cp = pltpu.make_async_copy(
                obuf.at[slot], out_hbm.at[:, sl, :], out_sems.at[slot])
            cp.start()
            out_copies[slot] = cp
        for cp in out_copies:
            cp.wait()

    return pl.pallas_call(
        body,
        out_shape=jax.ShapeDtypeStruct((b, s, c), jnp.bfloat16),
        in_specs=[
            pl.BlockSpec(memory_space=pl.ANY),
            pl.BlockSpec(memory_space=pltpu.VMEM),
            pl.BlockSpec(memory_space=pltpu.VMEM),
            pl.BlockSpec(memory_space=pltpu.VMEM),
        ],
        out_specs=pl.BlockSpec(memory_space=pl.ANY),
        scratch_shapes=[
            pltpu.VMEM((b, s, c), jnp.float32),
            pltpu.VMEM((2, b, ch, c), jnp.bfloat16),
            pltpu.VMEM((2, b, s), jnp.bfloat16),
            pltpu.VMEM((N_DEV - 1, 2, b, s), jnp.bfloat16),
            pltpu.SemaphoreType.DMA((N_CHUNKS,)),
            pltpu.SemaphoreType.DMA((2,)),
            pltpu.SemaphoreType.DMA((N_DEV - 1,)),
            pltpu.SemaphoreType.DMA((N_DEV - 1,)),
        ],
        compiler_params=pltpu.CompilerParams(collective_id=0),
    )(x, t_emb, W_scale, W_shift)


# device time: 22474 ns/iter; 1.0733x vs baseline; 1.0733x over previous
import jax
import jax.numpy as jnp
from jax import lax
from jax.experimental import pallas as pl
from jax.experimental.pallas import tpu as pltpu

N_DEV = 16
EPS = 1e-5
N_CHUNKS = 4
N_HALVES = 2


def kernel(x, t_emb, W_scale, W_shift):
    b, s, c = x.shape
    c_global = c * N_DEV
    ch = s // N_CHUNKS
    hs = s // N_HALVES
    cph = N_CHUNKS // N_HALVES

    def body(x_hbm, t_ref, ws_ref, wsh_ref, out_hbm,
             x_vmem, obuf, stats_ref, gather_ref,
             in_sems, out_sems, send_sems, recv_sems):
        my = lax.axis_index("i")

        in_copies = []
        for i in range(N_CHUNKS):
            cp = pltpu.make_async_copy(
                x_hbm.at[:, pl.ds(i * ch, ch), :],
                x_vmem.at[:, pl.ds(i * ch, ch), :],
                in_sems.at[i],
            )
            cp.start()
            in_copies.append(cp)

        barrier_sem = pltpu.get_barrier_semaphore()
        for k in range(1, N_DEV):
            pl.semaphore_signal(
                barrier_sem, inc=1,
                device_id=((my + k) % N_DEV,),
                device_id_type=pl.DeviceIdType.MESH,
            )
        pl.semaphore_wait(barrier_sem, N_DEV - 1)

        scale = jnp.dot(t_ref[...], ws_ref[...],
                        preferred_element_type=jnp.float32)
        shift = jnp.dot(t_ref[...], wsh_ref[...],
                        preferred_element_type=jnp.float32)
        mul = 1.0 + scale[:, None, :]
        add = shift[:, None, :]

        rdmas = [[], []]
        for i in range(N_CHUNKS):
            in_copies[i].wait()
            sl = slice(i * ch, (i + 1) * ch)
            xc = x_vmem[:, sl, :]
            stats_ref[0, :, sl] = jnp.sum(xc, axis=-1).astype(jnp.bfloat16)
            stats_ref[1, :, sl] = (
                jnp.sum(xc * xc, axis=-1).astype(jnp.bfloat16))
            if (i + 1) % cph == 0:
                h = i // cph
                hsl = pl.ds(h * hs, hs)
                for k in range(1, N_DEV):
                    rdma = pltpu.make_async_remote_copy(
                        src_ref=stats_ref.at[:, :, hsl],
                        dst_ref=gather_ref.at[k - 1, :, :, hsl],
                        send_sem=send_sems.at[h, k - 1],
                        recv_sem=recv_sems.at[h, k - 1],
                        device_id=((my + k) % N_DEV,),
                        device_id_type=pl.DeviceIdType.MESH,
                    )
                    rdma.start()
                    rdmas[h].append(rdma)

        out_copies = [None, None]
        for h in range(N_HALVES):
            for rdma in rdmas[h]:
                rdma.wait()
            hsl = slice(h * hs, (h + 1) * hs)
            tot = stats_ref[:, :, hsl].astype(jnp.float32) + jnp.sum(
                gather_ref[:, :, :, hsl].astype(jnp.float32), axis=0
            )
            mean = tot[0] / c_global
            inv = lax.rsqrt(tot[1] / c_global - mean * mean + EPS)

            for j in range(cph):
                i = h * cph + j
                slot = i % 2
                if out_copies[slot] is not None:
                    out_copies[slot].wait()
                sl = slice(i * ch, (i + 1) * ch)
                lsl = slice(j * ch, (j + 1) * ch)
                xc = x_vmem[:, sl, :]
                hn = (xc - mean[:, lsl, None]) * inv[:, lsl, None]
                obuf[slot] = (hn * mul + add).astype(jnp.bfloat16)
                cp = pltpu.make_async_copy(
                    obuf.at[slot], out_hbm.at[:, sl, :], out_sems.at[slot])
                cp.start()
                out_copies[slot] = cp
        for cp in out_copies:
            cp.wait()

    return pl.pallas_call(
        body,
        out_shape=jax.ShapeDtypeStruct((b, s, c), jnp.bfloat16),
        in_specs=[
            pl.BlockSpec(memory_space=pl.ANY),
            pl.BlockSpec(memory_space=pltpu.VMEM),
            pl.BlockSpec(memory_space=pltpu.VMEM),
            pl.BlockSpec(memory_space=pltpu.VMEM),
        ],
        out_specs=pl.BlockSpec(memory_space=pl.ANY),
        scratch_shapes=[
            pltpu.VMEM((b, s, c), jnp.float32),
            pltpu.VMEM((2, b, ch, c), jnp.bfloat16),
            pltpu.VMEM((2, b, s), jnp.bfloat16),
            pltpu.VMEM((N_DEV - 1, 2, b, s), jnp.bfloat16),
            pltpu.SemaphoreType.DMA((N_CHUNKS,)),
            pltpu.SemaphoreType.DMA((2,)),
            pltpu.SemaphoreType.DMA((N_HALVES, N_DEV - 1)),
            pltpu.SemaphoreType.DMA((N_HALVES, N_DEV - 1)),
        ],
        compiler_params=pltpu.CompilerParams(collective_id=0),
    )(x, t_emb, W_scale, W_shift)


# device time: 21490 ns/iter; 1.1224x vs baseline; 1.0458x over previous
import jax
import jax.numpy as jnp
from jax import lax
from jax.experimental import pallas as pl
from jax.experimental.pallas import tpu as pltpu

N_DEV = 16
EPS = 1e-5
N_CHUNKS = 4
N_HALVES = 2


def kernel(x, t_emb, W_scale, W_shift):
    b, s, c = x.shape
    c_global = c * N_DEV
    ch = s // N_CHUNKS
    hs = s // N_HALVES
    cph = N_CHUNKS // N_HALVES

    def body(x_hbm, t_ref, ws_ref, wsh_ref, out_hbm,
             x_vmem, obuf, stats_ref, gather_ref,
             in_sems, out_sems, send_sems, recv_sems):
        my = lax.axis_index("i")

        in_copies = []
        for i in range(N_CHUNKS):
            cp = pltpu.make_async_copy(
                x_hbm.at[:, pl.ds(i * ch, ch), :],
                x_vmem.at[:, pl.ds(i * ch, ch), :],
                in_sems.at[i],
            )
            cp.start()
            in_copies.append(cp)

        barrier_sem = pltpu.get_barrier_semaphore()
        for k in range(1, N_DEV):
            pl.semaphore_signal(
                barrier_sem, inc=1,
                device_id=((my + k) % N_DEV,),
                device_id_type=pl.DeviceIdType.MESH,
            )
        pl.semaphore_wait(barrier_sem, N_DEV - 1)

        scale = jnp.dot(t_ref[...], ws_ref[...],
                        preferred_element_type=jnp.float32)
        shift = jnp.dot(t_ref[...], wsh_ref[...],
                        preferred_element_type=jnp.float32)
        mul = 1.0 + scale[:, None, :]
        add = shift[:, None, :]

        rdmas = [[], []]
        for i in range(N_CHUNKS):
            in_copies[i].wait()
            sl = slice(i * ch, (i + 1) * ch)
            xc = x_vmem[:, sl, :]
            stats_ref[0, :, sl] = jnp.sum(xc, axis=-1).astype(jnp.bfloat16)
            stats_ref[1, :, sl] = (
                jnp.sum(xc * xc, axis=-1).astype(jnp.bfloat16))
            if (i + 1) % cph == 0:
                h = i // cph
                hsl = pl.ds(h * hs, hs)
                for k in range(1, N_DEV):
                    rdma = pltpu.make_async_remote_copy(
                        src_ref=stats_ref.at[:, :, hsl],
                        dst_ref=gather_ref.at[k - 1, :, :, hsl],
                        send_sem=send_sems.at[h, k - 1],
                        recv_sem=recv_sems.at[h, k - 1],
                        device_id=((my + k) % N_DEV,),
                        device_id_type=pl.DeviceIdType.MESH,
                    )
                    rdma.start()
                    rdmas[h].append(rdma)

        out_copies = [None, None]
        for h in range(N_HALVES):
            for rdma in rdmas[h]:
                rdma.wait()
            hsl = slice(h * hs, (h + 1) * hs)
            tot = stats_ref[:, :, hsl].astype(jnp.float32) + jnp.sum(
                gather_ref[:, :, :, hsl].astype(jnp.float32), axis=0
            )
            mean = tot[0] / c_global
            inv = lax.rsqrt(tot[1] / c_global - mean * mean + EPS)

            for j in range(cph):
                i = h * cph + j
                slot = i % 2
                if out_copies[slot] is not None:
                    out_copies[slot].wait()
                sl = slice(i * ch, (i + 1) * ch)
                lsl = slice(j * ch, (j + 1) * ch)
                xc = x_vmem[:, sl, :]
                hn = (xc - mean[:, lsl, None]) * inv[:, lsl, None]
                obuf[slot] = (hn * mul + add).astype(jnp.bfloat16)
                cp = pltpu.make_async_copy(
                    obuf.at[slot], out_hbm.at[:, sl, :], out_sems.at[slot])
                cp.start()
                out_copies[slot] = cp
        for cp in out_copies:
            cp.wait()

    return pl.pallas_call(
        body,
        out_shape=jax.ShapeDtypeStruct((b, s, c), jnp.bfloat16),
        in_specs=[
            pl.BlockSpec(memory_space=pltpu.MemorySpace.HBM),
            pl.BlockSpec(memory_space=pltpu.VMEM),
            pl.BlockSpec(memory_space=pltpu.VMEM),
            pl.BlockSpec(memory_space=pltpu.VMEM),
        ],
        out_specs=pl.BlockSpec(memory_space=pltpu.MemorySpace.HBM),
        scratch_shapes=[
            pltpu.VMEM((b, s, c), jnp.float32),
            pltpu.VMEM((2, b, ch, c), jnp.bfloat16),
            pltpu.VMEM((2, b, s), jnp.bfloat16),
            pltpu.VMEM((N_DEV - 1, 2, b, s), jnp.bfloat16),
            pltpu.SemaphoreType.DMA((N_CHUNKS,)),
            pltpu.SemaphoreType.DMA((2,)),
            pltpu.SemaphoreType.DMA((N_HALVES, N_DEV - 1)),
            pltpu.SemaphoreType.DMA((N_HALVES, N_DEV - 1)),
        ],
        compiler_params=pltpu.CompilerParams(collective_id=0),
    )(x, t_emb, W_scale, W_shift)


# device time: 20114 ns/iter; 1.1992x vs baseline; 1.0684x over previous
import jax
import jax.numpy as jnp
from jax import lax
from jax.experimental import pallas as pl
from jax.experimental.pallas import tpu as pltpu

N_DEV = 16
EPS = 1e-5
N_HALVES = 2


def kernel(x, t_emb, W_scale, W_shift):
    b, s, c = x.shape
    c_global = c * N_DEV
    hs = s // N_HALVES

    def body(x_ref, t_ref, ws_ref, wsh_ref, out_ref,
             stats_ref, gather_ref, send_sems, recv_sems):
        my = lax.axis_index("i")

        barrier_sem = pltpu.get_barrier_semaphore()
        for k in range(1, N_DEV):
            pl.semaphore_signal(
                barrier_sem, inc=1,
                device_id=((my + k) % N_DEV,),
                device_id_type=pl.DeviceIdType.MESH,
            )

        rdmas = [[], []]
        for h in range(N_HALVES):
            hsl = slice(h * hs, (h + 1) * hs)
            xc = x_ref[:, hsl, :]
            stats_ref[0, :, hsl] = jnp.sum(xc, axis=-1).astype(jnp.bfloat16)
            stats_ref[1, :, hsl] = (
                jnp.sum(xc * xc, axis=-1).astype(jnp.bfloat16))
            if h == 0:
                pl.semaphore_wait(barrier_sem, N_DEV - 1)
            for k in range(1, N_DEV):
                rdma = pltpu.make_async_remote_copy(
                    src_ref=stats_ref.at[:, :, pl.ds(h * hs, hs)],
                    dst_ref=gather_ref.at[k - 1, :, :, pl.ds(h * hs, hs)],
                    send_sem=send_sems.at[h, k - 1],
                    recv_sem=recv_sems.at[h, k - 1],
                    device_id=((my + k) % N_DEV,),
                    device_id_type=pl.DeviceIdType.MESH,
                )
                rdma.start()
                rdmas[h].append(rdma)

        scale = jnp.dot(t_ref[...], ws_ref[...],
                        preferred_element_type=jnp.float32)
        shift = jnp.dot(t_ref[...], wsh_ref[...],
                        preferred_element_type=jnp.float32)
        mul = 1.0 + scale[:, None, :]
        add = shift[:, None, :]

        for h in range(N_HALVES):
            for rdma in rdmas[h]:
                rdma.wait()
            hsl = slice(h * hs, (h + 1) * hs)
            tot = stats_ref[:, :, hsl].astype(jnp.float32) + jnp.sum(
                gather_ref[:, :, :, hsl].astype(jnp.float32), axis=0
            )
            mean = tot[0] / c_global
            inv = lax.rsqrt(tot[1] / c_global - mean * mean + EPS)

            xc = x_ref[:, hsl, :]
            hn = (xc - mean[:, :, None]) * inv[:, :, None]
            out_ref[:, hsl, :] = (hn * mul + add).astype(jnp.bfloat16)

    return pl.pallas_call(
        body,
        out_shape=jax.ShapeDtypeStruct((b, s, c), jnp.bfloat16),
        in_specs=[
            pl.BlockSpec(memory_space=pltpu.VMEM),
            pl.BlockSpec(memory_space=pltpu.VMEM),
            pl.BlockSpec(memory_space=pltpu.VMEM),
            pl.BlockSpec(memory_space=pltpu.VMEM),
        ],
        out_specs=pl.BlockSpec(memory_space=pltpu.VMEM),
        scratch_shapes=[
            pltpu.VMEM((2, b, s), jnp.bfloat16),
            pltpu.VMEM((N_DEV - 1, 2, b, s), jnp.bfloat16),
            pltpu.SemaphoreType.DMA((N_HALVES, N_DEV - 1)),
            pltpu.SemaphoreType.DMA((N_HALVES, N_DEV - 1)),
        ],
        compiler_params=pltpu.CompilerParams(collective_id=0),
    )(x, t_emb, W_scale, W_shift)
